# baseline (speedup 1.0000x reference)
import jax
import jax.numpy as jnp
from jax.experimental import pallas as pl
from jax.experimental.pallas import tpu as pltpu

CHUNK = 256
NBUF = 8
GROUP = 4


def _router_body(x_hbm, w_ref, b_ref, o_ref, buf, sems):
    rows = x_hbm.shape[0]
    nchunks = rows // CHUNK
    grows = GROUP * CHUNK
    ngroups = rows // grows

    def copy(i, slot):
        return pltpu.make_async_copy(
            x_hbm.at[pl.ds(i * CHUNK, CHUNK), :],
            buf.at[pl.ds(slot * CHUNK, CHUNK), :],
            sems.at[slot],
        )

    for s in range(NBUF):
        copy(s, s).start()

    def round_body(r, _):
        for half in range(NBUF // GROUP):
            g = r * (NBUF // GROUP) + half
            base = half * GROUP
            for s in range(GROUP):
                copy(g * GROUP + s, base + s).wait()
            xt = buf[pl.ds(base * CHUNK, grows), :]
            logits = jax.lax.dot_general(
                xt, w_ref[...],
                dimension_numbers=(((1,), (1,)), ((), ())),
                preferred_element_type=jnp.float32,
            ) + b_ref[...]
            m = jnp.max(logits, axis=-1, keepdims=True)
            e = jnp.exp(logits - m)
            p = e / jnp.sum(e, axis=-1, keepdims=True)
            o_ref[pl.ds(g * grows, grows), :] = p
            for s in range(GROUP):
                n = g * GROUP + NBUF + s

                @pl.when(n < nchunks)
                def _():
                    copy(n, base + s).start()
        return _

    jax.lax.fori_loop(0, ngroups // (NBUF // GROUP), round_body, None)


def kernel(x, W, b):
    B, T, D = x.shape
    E = W.shape[0]
    rows = B * T
    x2 = x.reshape(rows, D)
    out = pl.pallas_call(
        _router_body,
        in_specs=[
            pl.BlockSpec(memory_space=pltpu.MemorySpace.HBM),
            pl.BlockSpec(memory_space=pltpu.MemorySpace.VMEM),
            pl.BlockSpec(memory_space=pltpu.MemorySpace.VMEM),
        ],
        out_specs=pl.BlockSpec(memory_space=pltpu.MemorySpace.VMEM),
        out_shape=jax.ShapeDtypeStruct((rows, E), jnp.float32),
        scratch_shapes=[
            pltpu.VMEM((NBUF * CHUNK, 4096), jnp.float32),
            pltpu.SemaphoreType.DMA((NBUF,)),
        ],
    )(x2, W, b)
    return out.reshape(B, T, E)


# speedup vs baseline: 1.0033x; 1.0033x over previous
import jax
import jax.numpy as jnp
from jax.experimental import pallas as pl
from jax.experimental.pallas import tpu as pltpu

CHUNK = 256
NBUF = 8
GROUP = 4


def _router_body(x_hbm, w_ref, b_ref, o_ref, buf, sems):
    rows = x_hbm.shape[0]
    nchunks = rows // CHUNK
    grows = GROUP * CHUNK
    ngroups = rows // grows

    def copy(i, slot, sem):
        return pltpu.make_async_copy(
            x_hbm.at[pl.ds(i * CHUNK, CHUNK), :],
            buf.at[pl.ds(slot * CHUNK, CHUNK), :],
            sems.at[sem],
        )

    def group_wait(g, base, sem):
        pltpu.make_async_copy(
            x_hbm.at[pl.ds(g * grows, grows), :],
            buf.at[pl.ds(base * CHUNK, grows), :],
            sems.at[sem],
        ).wait()

    for s in range(NBUF):
        copy(s, s, s // GROUP).start()

    def round_body(r, _):
        for half in range(NBUF // GROUP):
            g = r * (NBUF // GROUP) + half
            base = half * GROUP
            group_wait(g, base, half)
            xt = buf[pl.ds(base * CHUNK, grows), :]
            logits = jax.lax.dot_general(
                xt, w_ref[...],
                dimension_numbers=(((1,), (1,)), ((), ())),
                preferred_element_type=jnp.float32,
            ) + b_ref[...]
            m = jnp.max(logits, axis=-1, keepdims=True)
            e = jnp.exp(logits - m)
            p = e / jnp.sum(e, axis=-1, keepdims=True)
            o_ref[pl.ds(g * grows, grows), :] = p
            for s in range(GROUP):
                n = g * GROUP + NBUF + s

                @pl.when(n < nchunks)
                def _():
                    copy(n, base + s, half).start()
        return _

    jax.lax.fori_loop(0, ngroups // (NBUF // GROUP), round_body, None)


def kernel(x, W, b):
    B, T, D = x.shape
    E = W.shape[0]
    rows = B * T
    x2 = x.reshape(rows, D)
    out = pl.pallas_call(
        _router_body,
        in_specs=[
            pl.BlockSpec(memory_space=pltpu.MemorySpace.HBM),
            pl.BlockSpec(memory_space=pltpu.MemorySpace.VMEM),
            pl.BlockSpec(memory_space=pltpu.MemorySpace.VMEM),
        ],
        out_specs=pl.BlockSpec(memory_space=pltpu.MemorySpace.VMEM),
        out_shape=jax.ShapeDtypeStruct((rows, E), jnp.float32),
        scratch_shapes=[
            pltpu.VMEM((NBUF * CHUNK, 4096), jnp.float32),
            pltpu.SemaphoreType.DMA((NBUF // GROUP,)),
        ],
    )(x2, W, b)
    return out.reshape(B, T, E)


# speedup vs baseline: 1.0615x; 1.0580x over previous
import jax
import jax.numpy as jnp
from jax.experimental import pallas as pl
from jax.experimental.pallas import tpu as pltpu

SUB = 256
NSUB = 4
TILE_M = SUB * NSUB


def _router_tile(x0, x1, x2, x3, w_ref, b_ref, o_ref):
    for k, xr in enumerate((x0, x1, x2, x3)):
        logits = jax.lax.dot_general(
            xr[...], w_ref[...],
            dimension_numbers=(((1,), (1,)), ((), ())),
            preferred_element_type=jnp.float32,
        ) + b_ref[...]
        m = jnp.max(logits, axis=-1, keepdims=True)
        e = jnp.exp(logits - m)
        o_ref[k * SUB:(k + 1) * SUB, :] = e / jnp.sum(e, axis=-1, keepdims=True)


def kernel(x, W, b):
    B, T, D = x.shape
    E = W.shape[0]
    rows = B * T
    x2 = x.reshape(rows, D)
    grid = (rows // TILE_M,)

    def sub_spec(k):
        return pl.BlockSpec((SUB, D), lambda i, k=k: (NSUB * i + k, 0))

    out = pl.pallas_call(
        _router_tile,
        grid=grid,
        in_specs=[
            sub_spec(0), sub_spec(1), sub_spec(2), sub_spec(3),
            pl.BlockSpec((E, D), lambda i: (0, 0)),
            pl.BlockSpec((E,), lambda i: (0,)),
        ],
        out_specs=pl.BlockSpec((TILE_M, E), lambda i: (i, 0)),
        out_shape=jax.ShapeDtypeStruct((rows, E), jnp.float32),
    )(x2, x2, x2, x2, W, b)
    return out.reshape(B, T, E)


# speedup vs baseline: 1.0637x; 1.0021x over previous
import jax
import jax.numpy as jnp
from jax.experimental import pallas as pl

TILE_M = 1024


def _router_tile(x_ref, w_ref, b_ref, o_ref):
    logits = jax.lax.dot_general(
        x_ref[...], w_ref[...],
        dimension_numbers=(((1,), (1,)), ((), ())),
        preferred_element_type=jnp.float32,
    ) + b_ref[...]
    m = jnp.max(logits, axis=-1, keepdims=True)
    e = jnp.exp(logits - m)
    o_ref[...] = e / jnp.sum(e, axis=-1, keepdims=True)


def kernel(x, W, b):
    B, T, D = x.shape
    E = W.shape[0]
    rows = B * T
    x2 = x.reshape(rows, D)
    grid = (rows // TILE_M,)
    out = pl.pallas_call(
        _router_tile,
        grid=grid,
        in_specs=[
            pl.BlockSpec((TILE_M, D), lambda i: (i, 0)),
            pl.BlockSpec((E, D), lambda i: (0, 0)),
            pl.BlockSpec((E,), lambda i: (0,)),
        ],
        out_specs=pl.BlockSpec((TILE_M, E), lambda i: (i, 0)),
        out_shape=jax.ShapeDtypeStruct((rows, E), jnp.float32),
    )(x2, W, b)
    return out.reshape(B, T, E)
